# baseline (speedup 1.0000x reference)
import functools

import jax
import jax.numpy as jnp
from jax import lax
from jax.experimental import pallas as pl
from jax.experimental.pallas import tpu as pltpu
from jax.experimental.pallas import tpu_sc as plsc


def _make_embed(n_rows: int, vocab: int, dim: int):
    info = plsc.get_sparse_core_info()
    nw = info.num_cores * info.num_subcores
    assert n_rows % nw == 0
    b_per_w = n_rows // nw

    IDX_SEG = 128
    K = 4
    C = K * IDX_SEG
    assert b_per_w % C == 0
    n_chunks = b_per_w // C

    mesh = plsc.VectorSubcoreMesh(core_axis_name="c", subcore_axis_name="s")

    @functools.partial(
        pl.kernel,
        mesh=mesh,
        compiler_params=pltpu.CompilerParams(use_tc_tiling_on_sc=False),
        out_type=jax.ShapeDtypeStruct((n_rows, dim), jnp.float32),
        scratch_types=[
            pltpu.VMEM((2, C), jnp.int32),
            pltpu.VMEM((2, C, dim), jnp.float32),
            pltpu.SemaphoreType.DMA,
            pltpu.SemaphoreType.DMA,
            pltpu.SemaphoreType.DMA,
        ],
    )
    def embed(table_hbm, idx_hbm, out_hbm, idx_v, rows_v, isem, gsem, osem):
        wid = lax.axis_index("s") * info.num_cores + lax.axis_index("c")
        base = wid * b_per_w

        def load_idx(ci, slot):
            return pltpu.async_copy(
                idx_hbm.at[pl.ds(base + ci * C, C)], idx_v.at[slot], isem)

        def start_gathers(slot):
            for j in range(K):
                pltpu.async_copy(
                    table_hbm.at[idx_v.at[slot, pl.ds(j * IDX_SEG, IDX_SEG)]],
                    rows_v.at[slot, pl.ds(j * IDX_SEG, IDX_SEG)],
                    gsem)

        def drain_gathers(slot):
            for j in range(K):
                pltpu.make_async_copy(
                    table_hbm.at[idx_v.at[slot, pl.ds(j * IDX_SEG, IDX_SEG)]],
                    rows_v.at[slot, pl.ds(j * IDX_SEG, IDX_SEG)],
                    gsem).wait()

        def store_rows(ci, slot):
            return pltpu.async_copy(
                rows_v.at[slot], out_hbm.at[pl.ds(base + ci * C, C)], osem)

        load_idx(0, 0).wait()
        start_gathers(0)

        def body(ci, _):
            slot = lax.rem(ci, 2)
            nxt = 1 - slot
            io = load_idx(ci + 1, nxt)
            drain_gathers(slot)
            io.wait()
            start_gathers(nxt)
            st = store_rows(ci, slot)
            st.wait()
            return 0

        lax.fori_loop(0, n_chunks - 1, body, 0, unroll=False)

        last = n_chunks - 1
        slot = lax.rem(last, 2)
        drain_gathers(slot)
        store_rows(last, slot).wait()

    return embed


def kernel(x, W):
    B, L = x.shape
    V, D = W.shape
    n_rows = B * L
    embed = _make_embed(n_rows, V, D)
    out = embed(W, x.reshape(n_rows))
    return out.reshape(B, L, D)


# speedup vs baseline: 1.0026x; 1.0026x over previous
import functools

import jax
import jax.numpy as jnp
from jax import lax
from jax.experimental import pallas as pl
from jax.experimental.pallas import tpu as pltpu
from jax.experimental.pallas import tpu_sc as plsc

_SEG = 128
_R = 8


def _make_embed(n_rows: int, vocab: int, dim: int):
    info = plsc.get_sparse_core_info()
    nw = info.num_cores * info.num_subcores
    assert n_rows % (nw * _SEG * _R) == 0
    b_per_w = n_rows // nw
    n_seg = b_per_w // _SEG
    n_outer = n_seg // _R

    mesh = plsc.VectorSubcoreMesh(core_axis_name="c", subcore_axis_name="s")

    @functools.partial(
        pl.kernel,
        mesh=mesh,
        compiler_params=pltpu.CompilerParams(use_tc_tiling_on_sc=False),
        out_type=jax.ShapeDtypeStruct((n_rows, dim), jnp.float32),
        scratch_types=[
            pltpu.VMEM((b_per_w,), jnp.int32),
            pltpu.VMEM((_R, _SEG, dim), jnp.float32),
            pltpu.SemaphoreType.DMA,
        ] + [pltpu.SemaphoreType.DMA] * _R,
    )
    def embed(table_hbm, idx_hbm, out_hbm, idx_v, rows_v, isem, *sems):
        wid = lax.axis_index("s") * info.num_cores + lax.axis_index("c")
        base = wid * b_per_w

        pltpu.async_copy(idx_hbm.at[pl.ds(base, b_per_w)], idx_v, isem).wait()

        def gather(seg, s):
            pltpu.async_copy(
                table_hbm.at[idx_v.at[pl.ds(seg * _SEG, _SEG)]],
                rows_v.at[s], sems[s])

        def wait_slot(s):
            pltpu.make_async_copy(
                table_hbm.at[pl.ds(0, _SEG)], rows_v.at[s], sems[s]).wait()

        def store(seg, s):
            pltpu.async_copy(
                rows_v.at[s], out_hbm.at[pl.ds(base + seg * _SEG, _SEG)],
                sems[s])

        for s in range(_R):
            gather(s, s)

        def body(g, _):
            for s in range(_R):
                seg = g * _R + s
                wait_slot(s)
                store(seg, s)
                @pl.when(g < n_outer - 1)
                def _():
                    wait_slot(s)
                    gather(seg + _R, s)
            return 0

        lax.fori_loop(0, n_outer, body, 0, unroll=False)

        for s in range(_R):
            wait_slot(s)

    return embed


def kernel(x, W):
    B, L = x.shape
    V, D = W.shape
    n_rows = B * L
    embed = _make_embed(n_rows, V, D)
    out = embed(W, x.reshape(n_rows))
    return out.reshape(B, L, D)
